# baseline (speedup 1.0000x reference)
import jax
import jax.numpy as jnp
from jax.experimental import pallas as pl
from jax.experimental.pallas import tpu as pltpu

_B = 1024
_NF = 100000
_D = 64
_H1 = 64
_H2 = 32
_KT = 2048
_NB = (_NF + _KT - 1) // _KT
_NC = 2
_BC = _B // _NC
_EPS = 1e-5

_TDOT = (((0,), (1,)), ((), ()))


def _acc_kernel(x_ref, e_ref, lw_ref, a_ref, q_ref):
    k = pl.program_id(1)

    @pl.when(k == 0)
    def _init():
        a_ref[...] = jnp.zeros_like(a_ref)
        q_ref[...] = jnp.zeros_like(q_ref)

    def _accumulate(x, e, lw):
        aug = jnp.concatenate([e, lw], axis=1)
        a_ref[...] += jax.lax.dot_general(
            aug, x, _TDOT, preferred_element_type=jnp.float32)
        q_ref[...] += jax.lax.dot_general(
            e * e, x * x, _TDOT, preferred_element_type=jnp.float32)

    @pl.when(k < _NB - 1)
    def _full_tile():
        _accumulate(x_ref[...].astype(jnp.bfloat16),
                    e_ref[...].astype(jnp.bfloat16),
                    lw_ref[...].astype(jnp.bfloat16))

    @pl.when(k == _NB - 1)
    def _partial_tile():
        nvalid = _NF - (_NB - 1) * _KT
        lane = jax.lax.broadcasted_iota(jnp.int32, (1, _KT), 1)
        sub = jax.lax.broadcasted_iota(jnp.int32, (_KT, 1), 0)
        x = jnp.where(lane < nvalid, x_ref[...], 0.0).astype(jnp.bfloat16)
        e = jnp.where(sub < nvalid, e_ref[...], 0.0).astype(jnp.bfloat16)
        lw = jnp.where(sub < nvalid, lw_ref[...], 0.0).astype(jnp.bfloat16)
        _accumulate(x, e, lw)


def _bn_t(v, g, b):
    mu = jnp.mean(v, axis=1, keepdims=True)
    var = jnp.mean(jnp.square(v - mu), axis=1, keepdims=True)
    return (v - mu) / jnp.sqrt(var + _EPS) * g + b


def _tail_kernel(a_ref, q_ref, lb_ref, g0_ref, b0_ref,
                 w1_ref, b1_ref, g1_ref, bb1_ref,
                 w2_ref, b2_ref, g2_ref, bb2_ref, hw_ref, out_ref):
    se = a_ref[:_D, :]
    lin = a_ref[_D:_D + 1, :]
    bi = 0.5 * (se * se - q_ref[...])
    z = _bn_t(bi, g0_ref[...], b0_ref[...])
    z = jnp.dot(w1_ref[...], z,
                preferred_element_type=jnp.float32) + b1_ref[...]
    z = jax.nn.relu(_bn_t(z, g1_ref[...], bb1_ref[...]))
    z = jnp.dot(w2_ref[...], z,
                preferred_element_type=jnp.float32) + b2_ref[...]
    z = jax.nn.relu(_bn_t(z, g2_ref[...], bb2_ref[...]))
    y = jnp.sum(z * hw_ref[...], axis=0, keepdims=True)
    out_ref[...] = y + lin + lb_ref[...]


def kernel(feature_values, feature_embed, lin_w, lin_b, bn0_g, bn0_b,
           W1, b1, bn1_g, bn1_b, W2, b2, bn2_g, bn2_b, h_w):
    acc_a, acc_q = pl.pallas_call(
        _acc_kernel,
        grid=(_NC, _NB),
        in_specs=[
            pl.BlockSpec((_BC, _KT), lambda b, k: (b, k)),
            pl.BlockSpec((_KT, _D), lambda b, k: (k, 0)),
            pl.BlockSpec((_KT, 1), lambda b, k: (k, 0)),
        ],
        out_specs=[
            pl.BlockSpec((_D + 1, _BC), lambda b, k: (0, b)),
            pl.BlockSpec((_D, _BC), lambda b, k: (0, b)),
        ],
        out_shape=[
            jax.ShapeDtypeStruct((_D + 1, _B), jnp.float32),
            jax.ShapeDtypeStruct((_D, _B), jnp.float32),
        ],
        compiler_params=pltpu.CompilerParams(
            dimension_semantics=("parallel", "arbitrary"),
        ),
    )(feature_values, feature_embed, lin_w.reshape(_NF, 1))

    out = pl.pallas_call(
        _tail_kernel,
        out_shape=jax.ShapeDtypeStruct((1, _B), jnp.float32),
    )(acc_a, acc_q,
      lin_b.reshape(1, 1), bn0_g.reshape(_D, 1), bn0_b.reshape(_D, 1),
      W1, b1.reshape(_H1, 1), bn1_g.reshape(_H1, 1), bn1_b.reshape(_H1, 1),
      W2, b2.reshape(_H2, 1), bn2_g.reshape(_H2, 1), bn2_b.reshape(_H2, 1),
      h_w.reshape(_H2, 1))
    return out.reshape(_B)


# speedup vs baseline: 1.0901x; 1.0901x over previous
import jax
import jax.numpy as jnp
from jax.experimental import pallas as pl
from jax.experimental.pallas import tpu as pltpu

_B = 1024
_NF = 100000
_D = 64
_H1 = 64
_H2 = 32
_KT = 4096
_NB = (_NF + _KT - 1) // _KT
_NC = 1
_BC = _B // _NC
_EPS = 1e-5

_TDOT = (((0,), (1,)), ((), ()))


def _acc_kernel(x_ref, e_ref, lw_ref, a_ref, q_ref):
    k = pl.program_id(1)

    @pl.when(k == 0)
    def _init():
        a_ref[...] = jnp.zeros_like(a_ref)
        q_ref[...] = jnp.zeros_like(q_ref)

    def _accumulate(x, e, lw):
        aug = jnp.concatenate([e, lw], axis=1)
        a_ref[...] += jax.lax.dot_general(
            aug, x, _TDOT, preferred_element_type=jnp.float32)
        q_ref[...] += jax.lax.dot_general(
            e * e, x * x, _TDOT, preferred_element_type=jnp.float32)

    @pl.when(k < _NB - 1)
    def _full_tile():
        _accumulate(x_ref[...].astype(jnp.bfloat16),
                    e_ref[...].astype(jnp.bfloat16),
                    lw_ref[...].astype(jnp.bfloat16))

    @pl.when(k == _NB - 1)
    def _partial_tile():
        nvalid = _NF - (_NB - 1) * _KT
        lane = jax.lax.broadcasted_iota(jnp.int32, (1, _KT), 1)
        sub = jax.lax.broadcasted_iota(jnp.int32, (_KT, 1), 0)
        x = jnp.where(lane < nvalid, x_ref[...], 0.0).astype(jnp.bfloat16)
        e = jnp.where(sub < nvalid, e_ref[...], 0.0).astype(jnp.bfloat16)
        lw = jnp.where(sub < nvalid, lw_ref[...], 0.0).astype(jnp.bfloat16)
        _accumulate(x, e, lw)


def _bn_t(v, g, b):
    mu = jnp.mean(v, axis=1, keepdims=True)
    var = jnp.mean(jnp.square(v - mu), axis=1, keepdims=True)
    return (v - mu) / jnp.sqrt(var + _EPS) * g + b


def _tail_kernel(a_ref, q_ref, lb_ref, g0_ref, b0_ref,
                 w1_ref, b1_ref, g1_ref, bb1_ref,
                 w2_ref, b2_ref, g2_ref, bb2_ref, hw_ref, out_ref):
    se = a_ref[:_D, :]
    lin = a_ref[_D:_D + 1, :]
    bi = 0.5 * (se * se - q_ref[...])
    z = _bn_t(bi, g0_ref[...], b0_ref[...])
    z = jnp.dot(w1_ref[...], z,
                preferred_element_type=jnp.float32) + b1_ref[...]
    z = jax.nn.relu(_bn_t(z, g1_ref[...], bb1_ref[...]))
    z = jnp.dot(w2_ref[...], z,
                preferred_element_type=jnp.float32) + b2_ref[...]
    z = jax.nn.relu(_bn_t(z, g2_ref[...], bb2_ref[...]))
    y = jnp.sum(z * hw_ref[...], axis=0, keepdims=True)
    out_ref[...] = y + lin + lb_ref[...]


def kernel(feature_values, feature_embed, lin_w, lin_b, bn0_g, bn0_b,
           W1, b1, bn1_g, bn1_b, W2, b2, bn2_g, bn2_b, h_w):
    acc_a, acc_q = pl.pallas_call(
        _acc_kernel,
        grid=(_NC, _NB),
        in_specs=[
            pl.BlockSpec((_BC, _KT), lambda b, k: (b, k)),
            pl.BlockSpec((_KT, _D), lambda b, k: (k, 0)),
            pl.BlockSpec((_KT, 1), lambda b, k: (k, 0)),
        ],
        out_specs=[
            pl.BlockSpec((_D + 1, _BC), lambda b, k: (0, b)),
            pl.BlockSpec((_D, _BC), lambda b, k: (0, b)),
        ],
        out_shape=[
            jax.ShapeDtypeStruct((_D + 1, _B), jnp.float32),
            jax.ShapeDtypeStruct((_D, _B), jnp.float32),
        ],
        compiler_params=pltpu.CompilerParams(
            dimension_semantics=("parallel", "arbitrary"),
        ),
    )(feature_values, feature_embed, lin_w.reshape(_NF, 1))

    out = pl.pallas_call(
        _tail_kernel,
        out_shape=jax.ShapeDtypeStruct((1, _B), jnp.float32),
    )(acc_a, acc_q,
      lin_b.reshape(1, 1), bn0_g.reshape(_D, 1), bn0_b.reshape(_D, 1),
      W1, b1.reshape(_H1, 1), bn1_g.reshape(_H1, 1), bn1_b.reshape(_H1, 1),
      W2, b2.reshape(_H2, 1), bn2_g.reshape(_H2, 1), bn2_b.reshape(_H2, 1),
      h_w.reshape(_H2, 1))
    return out.reshape(_B)


# speedup vs baseline: 1.1160x; 1.0238x over previous
import jax
import jax.numpy as jnp
from jax.experimental import pallas as pl
from jax.experimental.pallas import tpu as pltpu

_B = 1024
_NF = 100000
_D = 64
_H1 = 64
_H2 = 32
_KT = 1024
_NS = 4
_FT = _NS * _KT
_NG = (_NF + _FT - 1) // _FT
_LAST = (_NF - 1) // _KT
_EPS = 1e-5

_TDOT = (((0,), (1,)), ((), ()))


def _acc_kernel(x0_ref, x1_ref, x2_ref, x3_ref, e_ref, lw_ref,
                a_ref, q_ref):
    k = pl.program_id(0)
    xs = (x0_ref, x1_ref, x2_ref, x3_ref)

    @pl.when(k == 0)
    def _init():
        a_ref[...] = jnp.zeros_like(a_ref)
        q_ref[...] = jnp.zeros_like(q_ref)

    def _accumulate(x, e, lw):
        aug = jnp.concatenate([e, lw], axis=1)
        a_ref[...] += jax.lax.dot_general(
            aug, x, _TDOT, preferred_element_type=jnp.float32)
        q_ref[...] += jax.lax.dot_general(
            e * e, x * x, _TDOT, preferred_element_type=jnp.float32)

    @pl.when(k < _NG - 1)
    def _full_step():
        for i in range(_NS):
            sl = pl.ds(i * _KT, _KT)
            _accumulate(xs[i][...].astype(jnp.bfloat16),
                        e_ref[sl, :].astype(jnp.bfloat16),
                        lw_ref[sl, :].astype(jnp.bfloat16))

    @pl.when(k == _NG - 1)
    def _last_step():
        for i in range(_NS):
            base = ((_NG - 1) * _NS + i) * _KT
            nvalid = _NF - base
            if nvalid <= 0:
                continue
            sl = pl.ds(i * _KT, _KT)
            x = xs[i][...]
            e = e_ref[sl, :]
            lw = lw_ref[sl, :]
            if nvalid < _KT:
                lane = jax.lax.broadcasted_iota(jnp.int32, (1, _KT), 1)
                sub = jax.lax.broadcasted_iota(jnp.int32, (_KT, 1), 0)
                x = jnp.where(lane < nvalid, x, 0.0)
                e = jnp.where(sub < nvalid, e, 0.0)
                lw = jnp.where(sub < nvalid, lw, 0.0)
            _accumulate(x.astype(jnp.bfloat16), e.astype(jnp.bfloat16),
                        lw.astype(jnp.bfloat16))


def _bn_t(v, g, b):
    mu = jnp.mean(v, axis=1, keepdims=True)
    var = jnp.mean(jnp.square(v - mu), axis=1, keepdims=True)
    return (v - mu) / jnp.sqrt(var + _EPS) * g + b


def _tail_kernel(a_ref, q_ref, lb_ref, g0_ref, b0_ref,
                 w1_ref, b1_ref, g1_ref, bb1_ref,
                 w2_ref, b2_ref, g2_ref, bb2_ref, hw_ref, out_ref):
    se = a_ref[:_D, :]
    lin = a_ref[_D:_D + 1, :]
    bi = 0.5 * (se * se - q_ref[...])
    z = _bn_t(bi, g0_ref[...], b0_ref[...])
    z = jnp.dot(w1_ref[...], z,
                preferred_element_type=jnp.float32) + b1_ref[...]
    z = jax.nn.relu(_bn_t(z, g1_ref[...], bb1_ref[...]))
    z = jnp.dot(w2_ref[...], z,
                preferred_element_type=jnp.float32) + b2_ref[...]
    z = jax.nn.relu(_bn_t(z, g2_ref[...], bb2_ref[...]))
    y = jnp.sum(z * hw_ref[...], axis=0, keepdims=True)
    out_ref[...] = y + lin + lb_ref[...]


def _x_spec(i):
    return pl.BlockSpec(
        (_B, _KT), lambda k, i=i: (0, jnp.minimum(_NS * k + i, _LAST)))


def kernel(feature_values, feature_embed, lin_w, lin_b, bn0_g, bn0_b,
           W1, b1, bn1_g, bn1_b, W2, b2, bn2_g, bn2_b, h_w):
    acc_a, acc_q = pl.pallas_call(
        _acc_kernel,
        grid=(_NG,),
        in_specs=[
            _x_spec(0), _x_spec(1), _x_spec(2), _x_spec(3),
            pl.BlockSpec((_FT, _D), lambda k: (k, 0)),
            pl.BlockSpec((_FT, 1), lambda k: (k, 0)),
        ],
        out_specs=[
            pl.BlockSpec((_D + 1, _B), lambda k: (0, 0)),
            pl.BlockSpec((_D, _B), lambda k: (0, 0)),
        ],
        out_shape=[
            jax.ShapeDtypeStruct((_D + 1, _B), jnp.float32),
            jax.ShapeDtypeStruct((_D, _B), jnp.float32),
        ],
        compiler_params=pltpu.CompilerParams(
            dimension_semantics=("arbitrary",),
        ),
    )(feature_values, feature_values, feature_values, feature_values,
      feature_embed, lin_w.reshape(_NF, 1))

    out = pl.pallas_call(
        _tail_kernel,
        out_shape=jax.ShapeDtypeStruct((1, _B), jnp.float32),
    )(acc_a, acc_q,
      lin_b.reshape(1, 1), bn0_g.reshape(_D, 1), bn0_b.reshape(_D, 1),
      W1, b1.reshape(_H1, 1), bn1_g.reshape(_H1, 1), bn1_b.reshape(_H1, 1),
      W2, b2.reshape(_H2, 1), bn2_g.reshape(_H2, 1), bn2_b.reshape(_H2, 1),
      h_w.reshape(_H2, 1))
    return out.reshape(_B)
